# Initial kernel scaffold; baseline (speedup 1.0000x reference)
import jax
import jax.numpy as jnp
from jax.experimental import pallas as pl


def kernel(Q, K, hyperedge_index_0, extended_index, Wq, bq, Wk, bk, Wv, bv, Wo, bo):
    raise NotImplementedError("write your pallas kernel here")


# speedup vs baseline: 18.6757x; 18.6757x over previous
import functools
import math

import jax
import jax.numpy as jnp
import numpy as np
from jax import lax
from jax.experimental import pallas as pl
from jax.experimental.pallas import tpu as pltpu
from jax.experimental.pallas import tpu_sc as plsc

N = 10000
E = 320000
DIM = 128
H = 8
D_HEAD = DIM // H
M = 10000
INV_SQRT = 1.0 / math.sqrt(DIM)

NC, NS = 2, 16
NW = NC * NS
E_PER_W = E // NW
CH = 80
NCH = E_PER_W // CH
ROWS_PER_TILE = 632
M_PAD = NS * ROWS_PER_TILE
E_PER_TILE = E // NS
HW = DIM // 2


def _proj_body(q_ref, k_ref, wq_ref, bq_ref, wk_ref, bk_ref, wv_ref, bv_ref,
               qp_ref, kp_ref, vp_ref):
    dn = (((1,), (1,)), ((), ()))
    q = q_ref[...]
    k = k_ref[...]
    qp_ref[...] = lax.dot_general(q, wq_ref[...], dn,
                                  preferred_element_type=jnp.float32) + bq_ref[...]
    kp_ref[...] = lax.dot_general(k, wk_ref[...], dn,
                                  preferred_element_type=jnp.float32) + bk_ref[...]
    vp_ref[...] = lax.dot_general(k, wv_ref[...], dn,
                                  preferred_element_type=jnp.float32) + bv_ref[...]


def _edge_body(qi_ref, kj_ref, vj_ref, hmask_ref, sel_ref,
               t0_ref, t1_ref, e16_ref):
    prod = qi_ref[...] * kj_ref[...]
    dots = lax.dot_general(prod, hmask_ref[...], (((1,), (0,)), ((), ())),
                           preferred_element_type=jnp.float32)
    ex = jnp.exp(dots * INV_SQRT)
    t = ex * vj_ref[...]
    t0_ref[...] = t[:, :HW]
    t1_ref[...] = t[:, HW:]
    e16_ref[...] = lax.dot_general(ex, sel_ref[...], (((1,), (0,)), ((), ())),
                                   preferred_element_type=jnp.float32)


def _final_body(sp0_ref, sp1_ref, dp_ref, qhe_ref, pm_ref, expand_ref,
                wo_ref, bo_ref, o_ref):
    s = jnp.concatenate([sp0_ref[...], sp1_ref[...]], axis=1)
    d16 = dp_ref[...]
    d128 = lax.dot_general(d16, expand_ref[...], (((1,), (0,)), ((), ())),
                           preferred_element_type=jnp.float32)
    safe = jnp.where(d128 > 0.0, d128, 1.0)
    qkv = jnp.where(d128 > 0.0, s / safe, 0.0)
    x = qhe_ref[...] + qkv
    o_pre = lax.dot_general(x, pm_ref[...], (((1,), (0,)), ((), ())),
                            preferred_element_type=jnp.float32)
    lin = lax.dot_general(o_pre, wo_ref[...], (((1,), (1,)), ((), ())),
                          preferred_element_type=jnp.float32) + bo_ref[...]
    o_ref[...] = o_pre + jnp.maximum(lin, 0.0)


def _sc_gather(qp_hbm, kp_hbm, vp_hbm, eq_hbm, ek_hbm, he_hbm,
               qi_out, kj_out, vj_out, qhe_out,
               idx_v, rows_v, sem):
    c = lax.axis_index("c")
    s = lax.axis_index("s")
    wid = s * NC + c
    base0 = wid * E_PER_W

    def chunk(i, carry):
        base = pl.multiple_of(base0 + i * CH, 8)
        pltpu.sync_copy(eq_hbm.at[pl.ds(base, CH)], idx_v)
        pltpu.async_copy(qp_hbm.at[idx_v], rows_v, sem).wait()
        pltpu.sync_copy(rows_v, qi_out.at[pl.ds(base, CH)])
        pltpu.sync_copy(ek_hbm.at[pl.ds(base, CH)], idx_v)
        pltpu.async_copy(kp_hbm.at[idx_v], rows_v, sem).wait()
        pltpu.sync_copy(rows_v, kj_out.at[pl.ds(base, CH)])
        pltpu.async_copy(vp_hbm.at[idx_v], rows_v, sem).wait()
        pltpu.sync_copy(rows_v, vj_out.at[pl.ds(base, CH)])
        return carry

    lax.fori_loop(0, NCH, chunk, 0)

    @pl.when(wid < 25)
    def _():
        def hchunk(i, carry):
            base = pl.multiple_of(wid * 400 + i * CH, 8)
            pltpu.sync_copy(he_hbm.at[pl.ds(base, CH)], idx_v)
            pltpu.async_copy(qp_hbm.at[idx_v], rows_v, sem).wait()
            pltpu.sync_copy(rows_v, qhe_out.at[pl.ds(base, CH)])
            return carry
        lax.fori_loop(0, 400 // CH, hchunk, 0)


def _sc_scatter_sd(t_hbm, e16_hbm, le_hbm, z64_hbm, z16_hbm,
                   s_out, d_out,
                   idx_v, t_v, e_v, zb_v, zb16_v, s_sh, d_sh, sem):
    s = lax.axis_index("s")
    pltpu.sync_copy(z64_hbm, zb_v)
    pltpu.sync_copy(z16_hbm, zb16_v)
    row0 = s * ROWS_PER_TILE
    pltpu.sync_copy(zb_v, s_sh.at[pl.ds(row0, ROWS_PER_TILE)])
    pltpu.sync_copy(zb16_v, d_sh.at[pl.ds(row0, ROWS_PER_TILE)])
    plsc.subcore_barrier()

    base0 = s * E_PER_TILE

    def chunk(i, carry):
        base = pl.multiple_of(base0 + i * CH, 8)
        pltpu.sync_copy(le_hbm.at[pl.ds(base, CH)], idx_v)
        pltpu.sync_copy(t_hbm.at[pl.ds(base, CH)], t_v)
        pltpu.sync_copy(e16_hbm.at[pl.ds(base, CH)], e_v)
        pltpu.sync_copy(t_v, s_sh.at[idx_v], add=True)
        pltpu.sync_copy(e_v, d_sh.at[idx_v], add=True)
        return carry

    lax.fori_loop(0, E_PER_TILE // CH, chunk, 0)
    plsc.subcore_barrier()

    pltpu.sync_copy(s_sh.at[pl.ds(row0, ROWS_PER_TILE)], zb_v)
    pltpu.sync_copy(zb_v, s_out.at[pl.ds(row0, ROWS_PER_TILE)])
    pltpu.sync_copy(d_sh.at[pl.ds(row0, ROWS_PER_TILE)], zb16_v)
    pltpu.sync_copy(zb16_v, d_out.at[pl.ds(row0, ROWS_PER_TILE)])


def _sc_scatter_s(t_hbm, le_hbm, z64_hbm, s_out,
                  idx_v, t_v, zb_v, s_sh, sem):
    s = lax.axis_index("s")
    pltpu.sync_copy(z64_hbm, zb_v)
    row0 = s * ROWS_PER_TILE
    pltpu.sync_copy(zb_v, s_sh.at[pl.ds(row0, ROWS_PER_TILE)])
    plsc.subcore_barrier()

    base0 = s * E_PER_TILE

    def chunk(i, carry):
        base = pl.multiple_of(base0 + i * CH, 8)
        pltpu.sync_copy(le_hbm.at[pl.ds(base, CH)], idx_v)
        pltpu.sync_copy(t_hbm.at[pl.ds(base, CH)], t_v)
        pltpu.sync_copy(t_v, s_sh.at[idx_v], add=True)
        return carry

    lax.fori_loop(0, E_PER_TILE // CH, chunk, 0)
    plsc.subcore_barrier()

    pltpu.sync_copy(s_sh.at[pl.ds(row0, ROWS_PER_TILE)], zb_v)
    pltpu.sync_copy(zb_v, s_out.at[pl.ds(row0, ROWS_PER_TILE)])


def kernel(Q, K, hyperedge_index_0, extended_index, Wq, bq, Wk, bk, Wv, bv, Wo, bo):
    f32 = jnp.float32

    RB = 400
    bq2 = bq.reshape(1, DIM)
    bk2 = bk.reshape(1, DIM)
    bv2 = bv.reshape(1, DIM)
    row_spec = pl.BlockSpec((RB, DIM), lambda i: (i, 0))
    w_spec = pl.BlockSpec((DIM, DIM), lambda i: (0, 0))
    b_spec = pl.BlockSpec((1, DIM), lambda i: (0, 0))
    qp, kp, vp = pl.pallas_call(
        _proj_body,
        grid=(N // RB,),
        in_specs=[row_spec, row_spec, w_spec, b_spec, w_spec, b_spec, w_spec, b_spec],
        out_specs=[row_spec, row_spec, row_spec],
        out_shape=[jax.ShapeDtypeStruct((N, DIM), f32)] * 3,
    )(Q, K, Wq, bq2, Wk, bk2, Wv, bv2)

    eq = extended_index[:, 0].astype(jnp.int32)
    ek = extended_index[:, 1].astype(jnp.int32)
    le = extended_index[:, 3].astype(jnp.int32)
    he = hyperedge_index_0.astype(jnp.int32)

    mesh = plsc.VectorSubcoreMesh(core_axis_name="c", subcore_axis_name="s")
    gather_k = functools.partial(
        pl.kernel, mesh=mesh,
        out_type=[jax.ShapeDtypeStruct((E, DIM), f32)] * 3
        + [jax.ShapeDtypeStruct((M, DIM), f32)],
        scratch_types=[
            pltpu.VMEM((CH,), jnp.int32),
            pltpu.VMEM((CH, DIM), f32),
            pltpu.SemaphoreType.DMA,
        ],
    )(_sc_gather)
    qi, kj, vj, qhe = gather_k(qp, kp, vp, eq, ek, he)

    hmask = jnp.asarray(np.kron(np.eye(H, dtype=np.float32),
                                np.ones((D_HEAD, D_HEAD), np.float32)))
    sel = jnp.asarray(np.array([[1.0 if (j < H and c == D_HEAD * j) else 0.0
                                 for j in range(16)] for c in range(DIM)],
                               np.float32))
    EB = 1000
    e_row = pl.BlockSpec((EB, DIM), lambda i: (i, 0))
    e_half = pl.BlockSpec((EB, HW), lambda i: (i, 0))
    t0, t1, e16 = pl.pallas_call(
        _edge_body,
        grid=(E // EB,),
        in_specs=[e_row, e_row, e_row,
                  pl.BlockSpec((DIM, DIM), lambda i: (0, 0)),
                  pl.BlockSpec((DIM, 16), lambda i: (0, 0))],
        out_specs=[e_half, e_half, pl.BlockSpec((EB, 16), lambda i: (i, 0))],
        out_shape=[jax.ShapeDtypeStruct((E, HW), f32),
                   jax.ShapeDtypeStruct((E, HW), f32),
                   jax.ShapeDtypeStruct((E, 16), f32)],
    )(qi, kj, vj, hmask, sel)

    z64 = jnp.zeros((ROWS_PER_TILE, HW), f32)
    z16 = jnp.zeros((ROWS_PER_TILE, 16), f32)
    mesh1 = plsc.VectorSubcoreMesh(core_axis_name="c", subcore_axis_name="s",
                                   num_cores=1)
    cpar = pltpu.CompilerParams(use_tc_tiling_on_sc=False)
    scatter_sd_k = functools.partial(
        pl.kernel, mesh=mesh1, compiler_params=cpar,
        out_type=[jax.ShapeDtypeStruct((M_PAD, HW), f32),
                  jax.ShapeDtypeStruct((M_PAD, 16), f32)],
        scratch_types=[
            pltpu.VMEM((CH,), jnp.int32),
            pltpu.VMEM((CH, HW), f32),
            pltpu.VMEM((CH, 16), f32),
            pltpu.VMEM((ROWS_PER_TILE, HW), f32),
            pltpu.VMEM((ROWS_PER_TILE, 16), f32),
            pltpu.VMEM_SHARED((M_PAD, HW), f32),
            pltpu.VMEM_SHARED((M_PAD, 16), f32),
            pltpu.SemaphoreType.DMA,
        ],
    )(_sc_scatter_sd)
    s0_flat, d_flat = scatter_sd_k(t0, e16, le, z64, z16)

    scatter_s_k = functools.partial(
        pl.kernel, mesh=mesh1, compiler_params=cpar,
        out_type=jax.ShapeDtypeStruct((M_PAD, HW), f32),
        scratch_types=[
            pltpu.VMEM((CH,), jnp.int32),
            pltpu.VMEM((CH, HW), f32),
            pltpu.VMEM((ROWS_PER_TILE, HW), f32),
            pltpu.VMEM_SHARED((M_PAD, HW), f32),
            pltpu.SemaphoreType.DMA,
        ],
    )(_sc_scatter_s)
    s1_flat = scatter_s_k(t1, le, z64)

    sp0 = s0_flat[:M]
    sp1 = s1_flat[:M]
    dp = d_flat[:M]

    perm = np.zeros((DIM, DIM), np.float32)
    for h in range(H):
        for dd in range(D_HEAD):
            perm[h * D_HEAD + dd, dd * H + h] = 1.0
    pm = jnp.asarray(perm)
    expand = jnp.asarray(np.array(
        [[1.0 if (j < H and c // D_HEAD == j) else 0.0 for c in range(DIM)]
         for j in range(16)], np.float32))
    bo2 = bo.reshape(1, DIM)
    MB = 1000
    m_row = pl.BlockSpec((MB, DIM), lambda i: (i, 0))
    out = pl.pallas_call(
        _final_body,
        grid=(M // MB,),
        in_specs=[pl.BlockSpec((MB, HW), lambda i: (i, 0)),
                  pl.BlockSpec((MB, HW), lambda i: (i, 0)),
                  pl.BlockSpec((MB, 16), lambda i: (i, 0)),
                  m_row,
                  pl.BlockSpec((DIM, DIM), lambda i: (0, 0)),
                  pl.BlockSpec((16, DIM), lambda i: (0, 0)),
                  pl.BlockSpec((DIM, DIM), lambda i: (0, 0)),
                  pl.BlockSpec((1, DIM), lambda i: (0, 0))],
        out_specs=m_row,
        out_shape=jax.ShapeDtypeStruct((M, DIM), f32),
    )(sp0, sp1, dp, qhe, pm, expand, Wo, bo2)

    return out


# speedup vs baseline: 19.2585x; 1.0312x over previous
import functools
import math

import jax
import jax.numpy as jnp
import numpy as np
from jax import lax
from jax.experimental import pallas as pl
from jax.experimental.pallas import tpu as pltpu
from jax.experimental.pallas import tpu_sc as plsc

N = 10000
E = 320000
DIM = 128
H = 8
D_HEAD = DIM // H
M = 10000
INV_SQRT = 1.0 / math.sqrt(DIM)

NC, NS = 2, 16
NW = NC * NS
E_PER_W = E // NW
CH = 80
NCH = E_PER_W // CH
ROWS_PER_TILE = 632
M_PAD = NS * ROWS_PER_TILE
E_PER_TILE = E // NS
HW = DIM // 2


def _proj_body(q_ref, k_ref, wq_ref, bq_ref, wk_ref, bk_ref, wv_ref, bv_ref,
               qp_ref, kp_ref, vp_ref):
    dn = (((1,), (1,)), ((), ()))
    q = q_ref[...]
    k = k_ref[...]
    qp_ref[...] = lax.dot_general(q, wq_ref[...], dn,
                                  preferred_element_type=jnp.float32) + bq_ref[...]
    kp_ref[...] = lax.dot_general(k, wk_ref[...], dn,
                                  preferred_element_type=jnp.float32) + bk_ref[...]
    vp_ref[...] = lax.dot_general(k, wv_ref[...], dn,
                                  preferred_element_type=jnp.float32) + bv_ref[...]


def _edge_body(qi_ref, kj_ref, vj_ref, hmask_ref, sel_ref,
               ta_ref, t1_ref):
    prod = qi_ref[...] * kj_ref[...]
    dots = lax.dot_general(prod, hmask_ref[...], (((1,), (0,)), ((), ())),
                           preferred_element_type=jnp.float32)
    ex = jnp.exp(dots * INV_SQRT)
    t = ex * vj_ref[...]
    e16 = lax.dot_general(ex, sel_ref[...], (((1,), (0,)), ((), ())),
                          preferred_element_type=jnp.float32)
    ta_ref[...] = jnp.concatenate([t[:, :HW], e16], axis=1)
    t1_ref[...] = t[:, HW:]


def _final_body(sd_ref, sp1_ref, qhe_ref, pm_ref, expand_ref,
                wo_ref, bo_ref, o_ref):
    sd = sd_ref[...]
    s = jnp.concatenate([sd[:, :HW], sp1_ref[...]], axis=1)
    d16 = sd[:, HW:]
    d128 = lax.dot_general(d16, expand_ref[...], (((1,), (0,)), ((), ())),
                           preferred_element_type=jnp.float32)
    safe = jnp.where(d128 > 0.0, d128, 1.0)
    qkv = jnp.where(d128 > 0.0, s / safe, 0.0)
    x = qhe_ref[...] + qkv
    o_pre = lax.dot_general(x, pm_ref[...], (((1,), (0,)), ((), ())),
                            preferred_element_type=jnp.float32)
    lin = lax.dot_general(o_pre, wo_ref[...], (((1,), (1,)), ((), ())),
                          preferred_element_type=jnp.float32) + bo_ref[...]
    o_ref[...] = o_pre + jnp.maximum(lin, 0.0)


def _extract_col(ext_v, idx_v, col):
    lanes = lax.iota(jnp.int32, 16)
    for k in range(CH // 16):
        vals = plsc.load_gather(ext_v, [(lanes + 16 * k) * 4 + col])
        idx_v[pl.ds(16 * k, 16)] = vals


def _sc_gather(qp_hbm, kp_hbm, vp_hbm, ext_hbm, he_hbm,
               qi_out, kj_out, vj_out, qhe_out,
               ext_v, idx_v, rows_v, sem):
    c = lax.axis_index("c")
    s = lax.axis_index("s")
    wid = s * NC + c
    base0 = wid * E_PER_W

    def chunk(i, carry):
        base = pl.multiple_of(base0 + i * CH, 8)
        pltpu.sync_copy(ext_hbm.at[pl.ds(base * 4, CH * 4)], ext_v)
        _extract_col(ext_v, idx_v, 0)
        pltpu.async_copy(qp_hbm.at[idx_v], rows_v, sem).wait()
        pltpu.sync_copy(rows_v, qi_out.at[pl.ds(base, CH)])
        _extract_col(ext_v, idx_v, 1)
        pltpu.async_copy(kp_hbm.at[idx_v], rows_v, sem).wait()
        pltpu.sync_copy(rows_v, kj_out.at[pl.ds(base, CH)])
        pltpu.async_copy(vp_hbm.at[idx_v], rows_v, sem).wait()
        pltpu.sync_copy(rows_v, vj_out.at[pl.ds(base, CH)])
        return carry

    lax.fori_loop(0, NCH, chunk, 0)

    @pl.when(wid < 25)
    def _():
        def hchunk(i, carry):
            base = pl.multiple_of(wid * 400 + i * CH, 8)
            pltpu.sync_copy(he_hbm.at[pl.ds(base, CH)], idx_v)
            pltpu.async_copy(qp_hbm.at[idx_v], rows_v, sem).wait()
            pltpu.sync_copy(rows_v, qhe_out.at[pl.ds(base, CH)])
            return carry
        lax.fori_loop(0, 400 // CH, hchunk, 0)


def _make_scatter(width):
    def body(t_hbm, ext_hbm, z_hbm, s_out, ext_v, idx_v, t_v, zb_v, s_sh, sem):
        s = lax.axis_index("s")
        pltpu.sync_copy(z_hbm, zb_v)
        row0 = s * ROWS_PER_TILE
        pltpu.sync_copy(zb_v, s_sh.at[pl.ds(row0, ROWS_PER_TILE)])
        plsc.subcore_barrier()

        base0 = s * E_PER_TILE

        def chunk(i, carry):
            base = pl.multiple_of(base0 + i * CH, 8)
            pltpu.sync_copy(ext_hbm.at[pl.ds(base * 4, CH * 4)], ext_v)
            pltpu.sync_copy(t_hbm.at[pl.ds(base, CH)], t_v)
            _extract_col(ext_v, idx_v, 3)
            pltpu.sync_copy(t_v, s_sh.at[idx_v], add=True)
            return carry

        lax.fori_loop(0, E_PER_TILE // CH, chunk, 0)
        plsc.subcore_barrier()

        pltpu.sync_copy(s_sh.at[pl.ds(row0, ROWS_PER_TILE)], zb_v)
        pltpu.sync_copy(zb_v, s_out.at[pl.ds(row0, ROWS_PER_TILE)])

    return body


def kernel(Q, K, hyperedge_index_0, extended_index, Wq, bq, Wk, bk, Wv, bv, Wo, bo):
    f32 = jnp.float32

    RB = 400
    bq2 = bq.reshape(1, DIM)
    bk2 = bk.reshape(1, DIM)
    bv2 = bv.reshape(1, DIM)
    row_spec = pl.BlockSpec((RB, DIM), lambda i: (i, 0))
    w_spec = pl.BlockSpec((DIM, DIM), lambda i: (0, 0))
    b_spec = pl.BlockSpec((1, DIM), lambda i: (0, 0))
    qp, kp, vp = pl.pallas_call(
        _proj_body,
        grid=(N // RB,),
        in_specs=[row_spec, row_spec, w_spec, b_spec, w_spec, b_spec, w_spec, b_spec],
        out_specs=[row_spec, row_spec, row_spec],
        out_shape=[jax.ShapeDtypeStruct((N, DIM), f32)] * 3,
    )(Q, K, Wq, bq2, Wk, bk2, Wv, bv2)

    ext_flat = extended_index.astype(jnp.int32).reshape(-1)
    he = hyperedge_index_0.astype(jnp.int32)

    mesh = plsc.VectorSubcoreMesh(core_axis_name="c", subcore_axis_name="s")
    gather_k = functools.partial(
        pl.kernel, mesh=mesh,
        compiler_params=pltpu.CompilerParams(needs_layout_passes=False),
        out_type=[jax.ShapeDtypeStruct((E, DIM), f32)] * 3
        + [jax.ShapeDtypeStruct((M, DIM), f32)],
        scratch_types=[
            pltpu.VMEM((CH * 4,), jnp.int32),
            pltpu.VMEM((CH,), jnp.int32),
            pltpu.VMEM((CH, DIM), f32),
            pltpu.SemaphoreType.DMA,
        ],
    )(_sc_gather)
    qi, kj, vj, qhe = gather_k(qp, kp, vp, ext_flat, he)

    hmask = jnp.asarray(np.kron(np.eye(H, dtype=np.float32),
                                np.ones((D_HEAD, D_HEAD), np.float32)))
    sel = jnp.asarray(np.array([[1.0 if (j < H and c == D_HEAD * j) else 0.0
                                 for j in range(16)] for c in range(DIM)],
                               np.float32))
    EB = 1000
    e_row = pl.BlockSpec((EB, DIM), lambda i: (i, 0))
    ta, t1 = pl.pallas_call(
        _edge_body,
        grid=(E // EB,),
        in_specs=[e_row, e_row, e_row,
                  pl.BlockSpec((DIM, DIM), lambda i: (0, 0)),
                  pl.BlockSpec((DIM, 16), lambda i: (0, 0))],
        out_specs=[pl.BlockSpec((EB, HW + 16), lambda i: (i, 0)),
                   pl.BlockSpec((EB, HW), lambda i: (i, 0))],
        out_shape=[jax.ShapeDtypeStruct((E, HW + 16), f32),
                   jax.ShapeDtypeStruct((E, HW), f32)],
    )(qi, kj, vj, hmask, sel)

    WA = HW + 16
    z80 = jnp.zeros((ROWS_PER_TILE, WA), f32)
    z64 = jnp.zeros((ROWS_PER_TILE, HW), f32)
    mesh1 = plsc.VectorSubcoreMesh(core_axis_name="c", subcore_axis_name="s",
                                   num_cores=1)
    cpar = pltpu.CompilerParams(use_tc_tiling_on_sc=False,
                                needs_layout_passes=False)

    def make_scatter_call(width):
        return functools.partial(
            pl.kernel, mesh=mesh1, compiler_params=cpar,
            out_type=jax.ShapeDtypeStruct((M_PAD, width), f32),
            scratch_types=[
                pltpu.VMEM((CH * 4,), jnp.int32),
                pltpu.VMEM((CH,), jnp.int32),
                pltpu.VMEM((CH, width), f32),
                pltpu.VMEM((ROWS_PER_TILE, width), f32),
                pltpu.VMEM_SHARED((M_PAD, width), f32),
                pltpu.SemaphoreType.DMA,
            ],
        )(_make_scatter(width))

    sd_flat = make_scatter_call(WA)(ta, ext_flat, z80)
    s1_flat = make_scatter_call(HW)(t1, ext_flat, z64)

    perm = np.zeros((DIM, DIM), np.float32)
    for h in range(H):
        for dd in range(D_HEAD):
            perm[h * D_HEAD + dd, dd * H + h] = 1.0
    pm = jnp.asarray(perm)
    expand = jnp.asarray(np.array(
        [[1.0 if (j < H and c // D_HEAD == j) else 0.0 for c in range(DIM)]
         for j in range(16)], np.float32))
    bo2 = bo.reshape(1, DIM)
    MB = 1000
    m_row = pl.BlockSpec((MB, DIM), lambda i: (i, 0))
    out = pl.pallas_call(
        _final_body,
        grid=(M // MB,),
        in_specs=[pl.BlockSpec((MB, HW + 16), lambda i: (i, 0)),
                  pl.BlockSpec((MB, HW), lambda i: (i, 0)),
                  m_row,
                  pl.BlockSpec((DIM, DIM), lambda i: (0, 0)),
                  pl.BlockSpec((16, DIM), lambda i: (0, 0)),
                  pl.BlockSpec((DIM, DIM), lambda i: (0, 0)),
                  pl.BlockSpec((1, DIM), lambda i: (0, 0))],
        out_specs=m_row,
        out_shape=jax.ShapeDtypeStruct((M, DIM), f32),
    )(sd_flat, s1_flat, qhe, pm, expand, Wo, bo2)

    return out


# speedup vs baseline: 24.5588x; 1.2752x over previous
import functools
import math

import jax
import jax.numpy as jnp
import numpy as np
from jax import lax
from jax.experimental import pallas as pl
from jax.experimental.pallas import tpu as pltpu
from jax.experimental.pallas import tpu_sc as plsc

N = 10000
E = 320000
DIM = 128
H = 8
D_HEAD = DIM // H
M = 10000
INV_SQRT = 1.0 / math.sqrt(DIM)

NC, NS = 2, 16
NW = NC * NS
E_PER_W = E // NW
CH = 80
NCH = E_PER_W // CH
ROWS_PER_TILE = 632
M_PAD = NS * ROWS_PER_TILE
E_PER_TILE = E // NS
HW = DIM // 2


def _proj_body(q_ref, k_ref, wq_ref, bq_ref, wk_ref, bk_ref, wv_ref, bv_ref,
               qp_ref, kp_ref, vp_ref):
    dn = (((1,), (1,)), ((), ()))
    q = q_ref[...]
    k = k_ref[...]
    qp_ref[...] = lax.dot_general(q, wq_ref[...], dn,
                                  preferred_element_type=jnp.float32) + bq_ref[...]
    kp_ref[...] = lax.dot_general(k, wk_ref[...], dn,
                                  preferred_element_type=jnp.float32) + bk_ref[...]
    vp_ref[...] = lax.dot_general(k, wv_ref[...], dn,
                                  preferred_element_type=jnp.float32) + bv_ref[...]


def _edge_body(qi_ref, kj_ref, vj_ref, hmask_ref, sel_ref,
               ta_ref, t1_ref):
    prod = qi_ref[...] * kj_ref[...]
    dots = lax.dot_general(prod, hmask_ref[...], (((1,), (0,)), ((), ())),
                           preferred_element_type=jnp.float32)
    ex = jnp.exp(dots * INV_SQRT)
    t = ex * vj_ref[...]
    e16 = lax.dot_general(ex, sel_ref[...], (((1,), (0,)), ((), ())),
                          preferred_element_type=jnp.float32)
    ta_ref[...] = jnp.concatenate([t[:, :HW], e16], axis=1)
    t1_ref[...] = t[:, HW:]


def _final_body(sd_ref, sp1_ref, qhe_ref, pm_ref, expand_ref,
                wo_ref, bo_ref, o_ref):
    sd = sd_ref[...]
    s = jnp.concatenate([sd[:, :HW], sp1_ref[...]], axis=1)
    d16 = sd[:, HW:]
    d128 = lax.dot_general(d16, expand_ref[...], (((1,), (0,)), ((), ())),
                           preferred_element_type=jnp.float32)
    safe = jnp.where(d128 > 0.0, d128, 1.0)
    qkv = jnp.where(d128 > 0.0, s / safe, 0.0)
    x = qhe_ref[...] + qkv
    o_pre = lax.dot_general(x, pm_ref[...], (((1,), (0,)), ((), ())),
                            preferred_element_type=jnp.float32)
    lin = lax.dot_general(o_pre, wo_ref[...], (((1,), (1,)), ((), ())),
                          preferred_element_type=jnp.float32) + bo_ref[...]
    o_ref[...] = o_pre + jnp.maximum(lin, 0.0)


def _extract_col(ext_v, idx_v, col):
    lanes = lax.iota(jnp.int32, 16)
    for k in range(CH // 16):
        vals = plsc.load_gather(ext_v, [(lanes + 16 * k) * 4 + col])
        idx_v[pl.ds(16 * k, 16)] = vals


def _sc_gather(qp_hbm, kp_hbm, vp_hbm, ext_hbm, he_hbm,
               qi_out, kj_out, vj_out, qhe_out,
               ext_v, idx_v, rows_v, sem):
    c = lax.axis_index("c")
    s = lax.axis_index("s")
    wid = s * NC + c
    base0 = wid * E_PER_W

    def chunk(i, carry):
        base = pl.multiple_of(base0 + i * CH, 8)
        pltpu.sync_copy(ext_hbm.at[pl.ds(base * 4, CH * 4)], ext_v)
        _extract_col(ext_v, idx_v, 0)
        pltpu.async_copy(qp_hbm.at[idx_v], rows_v, sem).wait()
        pltpu.sync_copy(rows_v, qi_out.at[pl.ds(base, CH)])
        _extract_col(ext_v, idx_v, 1)
        pltpu.async_copy(kp_hbm.at[idx_v], rows_v, sem).wait()
        pltpu.sync_copy(rows_v, kj_out.at[pl.ds(base, CH)])
        pltpu.async_copy(vp_hbm.at[idx_v], rows_v, sem).wait()
        pltpu.sync_copy(rows_v, vj_out.at[pl.ds(base, CH)])
        return carry

    lax.fori_loop(0, NCH, chunk, 0)

    @pl.when(wid < 25)
    def _():
        def hchunk(i, carry):
            base = pl.multiple_of(wid * 400 + i * CH, 8)
            pltpu.sync_copy(he_hbm.at[pl.ds(base, CH)], idx_v)
            pltpu.async_copy(qp_hbm.at[idx_v], rows_v, sem).wait()
            pltpu.sync_copy(rows_v, qhe_out.at[pl.ds(base, CH)])
            return carry
        lax.fori_loop(0, 400 // CH, hchunk, 0)


def _make_scatter(width):
    def body(t_hbm, ext_hbm, z_hbm, s_out,
             ext0_v, ext1_v, idx_v, t0_v, t1_v, zb_v, s_sh,
             semE0, semE1, semT0, semT1, sem):
        s = lax.axis_index("s")
        pltpu.sync_copy(z_hbm, zb_v)
        row0 = s * ROWS_PER_TILE
        pltpu.sync_copy(zb_v, s_sh.at[pl.ds(row0, ROWS_PER_TILE)])
        plsc.subcore_barrier()

        base0 = s * E_PER_TILE
        ext_v = (ext0_v, ext1_v)
        t_v = (t0_v, t1_v)
        semE = (semE0, semE1)
        semT = (semT0, semT1)
        nch = E_PER_TILE // CH

        def start(i, b):
            base = pl.multiple_of(base0 + i * CH, 8)
            pltpu.async_copy(ext_hbm.at[pl.ds(base * 4, CH * 4)], ext_v[b],
                             semE[b])
            pltpu.async_copy(t_hbm.at[pl.ds(base, CH)], t_v[b], semT[b])

        def finish(i, b):
            base = pl.multiple_of(base0 + i * CH, 8)
            pltpu.make_async_copy(ext_hbm.at[pl.ds(base * 4, CH * 4)],
                                  ext_v[b], semE[b]).wait()
            pltpu.make_async_copy(t_hbm.at[pl.ds(base, CH)], t_v[b],
                                  semT[b]).wait()
            _extract_col(ext_v[b], idx_v, 3)
            pltpu.sync_copy(t_v[b], s_sh.at[idx_v], add=True)

        start(0, 0)

        def outer(g, carry):
            i = g * 2
            start(i + 1, 1)
            finish(i, 0)

            @pl.when(g < nch // 2 - 1)
            def _():
                start(i + 2, 0)

            finish(i + 1, 1)
            return carry

        lax.fori_loop(0, nch // 2, outer, 0)
        plsc.subcore_barrier()

        pltpu.sync_copy(s_sh.at[pl.ds(row0, ROWS_PER_TILE)], zb_v)
        pltpu.sync_copy(zb_v, s_out.at[pl.ds(row0, ROWS_PER_TILE)])

    return body


def kernel(Q, K, hyperedge_index_0, extended_index, Wq, bq, Wk, bk, Wv, bv, Wo, bo):
    f32 = jnp.float32

    RB = 400
    bq2 = bq.reshape(1, DIM)
    bk2 = bk.reshape(1, DIM)
    bv2 = bv.reshape(1, DIM)
    row_spec = pl.BlockSpec((RB, DIM), lambda i: (i, 0))
    w_spec = pl.BlockSpec((DIM, DIM), lambda i: (0, 0))
    b_spec = pl.BlockSpec((1, DIM), lambda i: (0, 0))
    qp, kp, vp = pl.pallas_call(
        _proj_body,
        grid=(N // RB,),
        in_specs=[row_spec, row_spec, w_spec, b_spec, w_spec, b_spec, w_spec, b_spec],
        out_specs=[row_spec, row_spec, row_spec],
        out_shape=[jax.ShapeDtypeStruct((N, DIM), f32)] * 3,
    )(Q, K, Wq, bq2, Wk, bk2, Wv, bv2)

    ext_flat = extended_index.astype(jnp.int32).reshape(-1)
    he = hyperedge_index_0.astype(jnp.int32)

    mesh = plsc.VectorSubcoreMesh(core_axis_name="c", subcore_axis_name="s")
    gather_k = functools.partial(
        pl.kernel, mesh=mesh,
        compiler_params=pltpu.CompilerParams(needs_layout_passes=False),
        out_type=[jax.ShapeDtypeStruct((E, DIM), f32)] * 3
        + [jax.ShapeDtypeStruct((M, DIM), f32)],
        scratch_types=[
            pltpu.VMEM((CH * 4,), jnp.int32),
            pltpu.VMEM((CH,), jnp.int32),
            pltpu.VMEM((CH, DIM), f32),
            pltpu.SemaphoreType.DMA,
        ],
    )(_sc_gather)
    qi, kj, vj, qhe = gather_k(qp, kp, vp, ext_flat, he)

    hmask = jnp.asarray(np.kron(np.eye(H, dtype=np.float32),
                                np.ones((D_HEAD, D_HEAD), np.float32)))
    sel = jnp.asarray(np.array([[1.0 if (j < H and c == D_HEAD * j) else 0.0
                                 for j in range(16)] for c in range(DIM)],
                               np.float32))
    EB = 1000
    e_row = pl.BlockSpec((EB, DIM), lambda i: (i, 0))
    ta, t1 = pl.pallas_call(
        _edge_body,
        grid=(E // EB,),
        in_specs=[e_row, e_row, e_row,
                  pl.BlockSpec((DIM, DIM), lambda i: (0, 0)),
                  pl.BlockSpec((DIM, 16), lambda i: (0, 0))],
        out_specs=[pl.BlockSpec((EB, HW + 16), lambda i: (i, 0)),
                   pl.BlockSpec((EB, HW), lambda i: (i, 0))],
        out_shape=[jax.ShapeDtypeStruct((E, HW + 16), f32),
                   jax.ShapeDtypeStruct((E, HW), f32)],
    )(qi, kj, vj, hmask, sel)

    WA = HW + 16
    z80 = jnp.zeros((ROWS_PER_TILE, WA), f32)
    z64 = jnp.zeros((ROWS_PER_TILE, HW), f32)
    mesh1 = plsc.VectorSubcoreMesh(core_axis_name="c", subcore_axis_name="s",
                                   num_cores=1)
    cpar = pltpu.CompilerParams(use_tc_tiling_on_sc=False,
                                needs_layout_passes=False)

    def make_scatter_call(width):
        return functools.partial(
            pl.kernel, mesh=mesh1, compiler_params=cpar,
            out_type=jax.ShapeDtypeStruct((M_PAD, width), f32),
            scratch_types=[
                pltpu.VMEM((CH * 4,), jnp.int32),
                pltpu.VMEM((CH * 4,), jnp.int32),
                pltpu.VMEM((CH,), jnp.int32),
                pltpu.VMEM((CH, width), f32),
                pltpu.VMEM((CH, width), f32),
                pltpu.VMEM((ROWS_PER_TILE, width), f32),
                pltpu.VMEM_SHARED((M_PAD, width), f32),
                pltpu.SemaphoreType.DMA,
                pltpu.SemaphoreType.DMA,
                pltpu.SemaphoreType.DMA,
                pltpu.SemaphoreType.DMA,
                pltpu.SemaphoreType.DMA,
            ],
        )(_make_scatter(width))

    sd_flat = make_scatter_call(WA)(ta, ext_flat, z80)
    s1_flat = make_scatter_call(HW)(t1, ext_flat, z64)

    perm = np.zeros((DIM, DIM), np.float32)
    for h in range(H):
        for dd in range(D_HEAD):
            perm[h * D_HEAD + dd, dd * H + h] = 1.0
    pm = jnp.asarray(perm)
    expand = jnp.asarray(np.array(
        [[1.0 if (j < H and c // D_HEAD == j) else 0.0 for c in range(DIM)]
         for j in range(16)], np.float32))
    bo2 = bo.reshape(1, DIM)
    MB = 1000
    m_row = pl.BlockSpec((MB, DIM), lambda i: (i, 0))
    out = pl.pallas_call(
        _final_body,
        grid=(M // MB,),
        in_specs=[pl.BlockSpec((MB, HW + 16), lambda i: (i, 0)),
                  pl.BlockSpec((MB, HW), lambda i: (i, 0)),
                  m_row,
                  pl.BlockSpec((DIM, DIM), lambda i: (0, 0)),
                  pl.BlockSpec((16, DIM), lambda i: (0, 0)),
                  pl.BlockSpec((DIM, DIM), lambda i: (0, 0)),
                  pl.BlockSpec((1, DIM), lambda i: (0, 0))],
        out_specs=m_row,
        out_shape=jax.ShapeDtypeStruct((M, DIM), f32),
    )(sd_flat, s1_flat, qhe, pm, expand, Wo, bo2)

    return out


# speedup vs baseline: 30.2347x; 1.2311x over previous
import functools
import math

import jax
import jax.numpy as jnp
import numpy as np
from jax import lax
from jax.experimental import pallas as pl
from jax.experimental.pallas import tpu as pltpu
from jax.experimental.pallas import tpu_sc as plsc

N = 10000
E = 320000
DIM = 128
H = 8
D_HEAD = DIM // H
M = 10000
INV_SQRT = 1.0 / math.sqrt(DIM)

NC, NS = 2, 16
NW = NC * NS
E_PER_W = E // NW
CH = 80
NCH = E_PER_W // CH
ROWS_PER_TILE = 632
M_PAD = NS * ROWS_PER_TILE
E_PER_TILE = E // NS
HW = DIM // 2


def _proj_body(q_ref, k_ref, wq_ref, bq_ref, wk_ref, bk_ref, wv_ref, bv_ref,
               qp_ref, kp_ref, vp_ref):
    dn = (((1,), (1,)), ((), ()))
    q = q_ref[...]
    k = k_ref[...]
    qp_ref[...] = lax.dot_general(q, wq_ref[...], dn,
                                  preferred_element_type=jnp.float32) + bq_ref[...]
    kp_ref[...] = lax.dot_general(k, wk_ref[...], dn,
                                  preferred_element_type=jnp.float32) + bk_ref[...]
    vp_ref[...] = lax.dot_general(k, wv_ref[...], dn,
                                  preferred_element_type=jnp.float32) + bv_ref[...]


def _edge_body(qi_ref, kj_ref, vj_ref, hmask_ref, sel_ref,
               ta_ref, t1_ref):
    prod = qi_ref[...] * kj_ref[...]
    dots = lax.dot_general(prod, hmask_ref[...], (((1,), (0,)), ((), ())),
                           preferred_element_type=jnp.float32)
    ex = jnp.exp(dots * INV_SQRT)
    t = ex * vj_ref[...]
    e16 = lax.dot_general(ex, sel_ref[...], (((1,), (0,)), ((), ())),
                          preferred_element_type=jnp.float32)
    ta_ref[...] = jnp.concatenate([t[:, :HW], e16], axis=1)
    t1_ref[...] = t[:, HW:]


def _final_body(sd_ref, sp1_ref, qhe_ref, pm_ref, expand_ref,
                wo_ref, bo_ref, o_ref):
    sd = sd_ref[...]
    s = jnp.concatenate([sd[:, :HW], sp1_ref[...]], axis=1)
    d16 = sd[:, HW:]
    d128 = lax.dot_general(d16, expand_ref[...], (((1,), (0,)), ((), ())),
                           preferred_element_type=jnp.float32)
    safe = jnp.where(d128 > 0.0, d128, 1.0)
    qkv = jnp.where(d128 > 0.0, s / safe, 0.0)
    x = qhe_ref[...] + qkv
    o_pre = lax.dot_general(x, pm_ref[...], (((1,), (0,)), ((), ())),
                            preferred_element_type=jnp.float32)
    lin = lax.dot_general(o_pre, wo_ref[...], (((1,), (1,)), ((), ())),
                          preferred_element_type=jnp.float32) + bo_ref[...]
    o_ref[...] = o_pre + jnp.maximum(lin, 0.0)


def _extract_col(ext_v, idx_v, col):
    lanes = lax.iota(jnp.int32, 16)
    for k in range(CH // 16):
        vals = plsc.load_gather(ext_v, [(lanes + 16 * k) * 4 + col])
        idx_v[pl.ds(16 * k, 16)] = vals


def _sc_gather(qp_hbm, kp_hbm, vp_hbm, ext_hbm, he_hbm,
               qi_out, kj_out, vj_out, qhe_out,
               ext0_v, ext1_v, iq0_v, iq1_v, ik0_v, ik1_v,
               rq0_v, rq1_v, rk0_v, rk1_v, rv0_v, rv1_v,
               semE0, semE1, semW0, semW1, sem):
    c = lax.axis_index("c")
    s = lax.axis_index("s")
    wid = s * NC + c
    base0 = wid * E_PER_W

    ext_v = (ext0_v, ext1_v)
    iq_v = (iq0_v, iq1_v)
    ik_v = (ik0_v, ik1_v)
    rq_v = (rq0_v, rq1_v)
    rk_v = (rk0_v, rk1_v)
    rv_v = (rv0_v, rv1_v)
    semE = (semE0, semE1)
    semW = (semW0, semW1)

    def start_ext(i, b):
        base = pl.multiple_of(base0 + i * CH, 8)
        pltpu.async_copy(ext_hbm.at[pl.ds(base * 4, CH * 4)], ext_v[b],
                         semE[b])

    def wait_writebacks(i2, b):
        base = pl.multiple_of(base0 + i2 * CH, 8)
        pltpu.make_async_copy(rq_v[b], qi_out.at[pl.ds(base, CH)],
                              semW[b]).wait()
        pltpu.make_async_copy(rk_v[b], kj_out.at[pl.ds(base, CH)],
                              semW[b]).wait()
        pltpu.make_async_copy(rv_v[b], vj_out.at[pl.ds(base, CH)],
                              semW[b]).wait()

    def proc(i, b, g):
        base = pl.multiple_of(base0 + i * CH, 8)
        pltpu.make_async_copy(ext_hbm.at[pl.ds(base * 4, CH * 4)], ext_v[b],
                              semE[b]).wait()
        _extract_col(ext_v[b], iq_v[b], 0)
        _extract_col(ext_v[b], ik_v[b], 1)

        @pl.when(g > 0)
        def _():
            wait_writebacks(i - 2, b)

        hq = pltpu.async_copy(qp_hbm.at[iq_v[b]], rq_v[b], sem)
        hk = pltpu.async_copy(kp_hbm.at[ik_v[b]], rk_v[b], sem)
        hv = pltpu.async_copy(vp_hbm.at[ik_v[b]], rv_v[b], sem)
        hq.wait()
        hk.wait()
        hv.wait()
        pltpu.async_copy(rq_v[b], qi_out.at[pl.ds(base, CH)], semW[b])
        pltpu.async_copy(rk_v[b], kj_out.at[pl.ds(base, CH)], semW[b])
        pltpu.async_copy(rv_v[b], vj_out.at[pl.ds(base, CH)], semW[b])

    start_ext(0, 0)

    def outer(g, carry):
        i = g * 2
        start_ext(i + 1, 1)
        proc(i, 0, g)

        @pl.when(g < NCH // 2 - 1)
        def _():
            start_ext(i + 2, 0)

        proc(i + 1, 1, g)
        return carry

    lax.fori_loop(0, NCH // 2, outer, 0)
    if NCH % 2:
        start_ext(NCH - 1, 0)
        proc(NCH - 1, 0, 1)
        wait_writebacks(NCH - 2, 1)
        wait_writebacks(NCH - 1, 0)
    else:
        wait_writebacks(NCH - 2, 0)
        wait_writebacks(NCH - 1, 1)

    @pl.when(wid < 25)
    def _():
        def hchunk(i, carry):
            base = pl.multiple_of(wid * 400 + i * CH, 8)
            pltpu.sync_copy(he_hbm.at[pl.ds(base, CH)], iq0_v)
            pltpu.async_copy(qp_hbm.at[iq0_v], rq0_v, sem).wait()
            pltpu.sync_copy(rq0_v, qhe_out.at[pl.ds(base, CH)])
            return carry
        lax.fori_loop(0, 400 // CH, hchunk, 0)


def _make_scatter(width):
    def body(t_hbm, ext_hbm, z_hbm, s_out,
             ext0_v, ext1_v, idx_v, t0_v, t1_v, zb_v, s_sh,
             semE0, semE1, semT0, semT1, sem):
        s = lax.axis_index("s")
        pltpu.sync_copy(z_hbm, zb_v)
        row0 = s * ROWS_PER_TILE
        pltpu.sync_copy(zb_v, s_sh.at[pl.ds(row0, ROWS_PER_TILE)])
        plsc.subcore_barrier()

        base0 = s * E_PER_TILE
        ext_v = (ext0_v, ext1_v)
        t_v = (t0_v, t1_v)
        semE = (semE0, semE1)
        semT = (semT0, semT1)
        nch = E_PER_TILE // CH

        def start(i, b):
            base = pl.multiple_of(base0 + i * CH, 8)
            pltpu.async_copy(ext_hbm.at[pl.ds(base * 4, CH * 4)], ext_v[b],
                             semE[b])
            pltpu.async_copy(t_hbm.at[pl.ds(base, CH)], t_v[b], semT[b])

        def finish(i, b):
            base = pl.multiple_of(base0 + i * CH, 8)
            pltpu.make_async_copy(ext_hbm.at[pl.ds(base * 4, CH * 4)],
                                  ext_v[b], semE[b]).wait()
            pltpu.make_async_copy(t_hbm.at[pl.ds(base, CH)], t_v[b],
                                  semT[b]).wait()
            _extract_col(ext_v[b], idx_v, 3)
            pltpu.sync_copy(t_v[b], s_sh.at[idx_v], add=True)

        start(0, 0)

        def outer(g, carry):
            i = g * 2
            start(i + 1, 1)
            finish(i, 0)

            @pl.when(g < nch // 2 - 1)
            def _():
                start(i + 2, 0)

            finish(i + 1, 1)
            return carry

        lax.fori_loop(0, nch // 2, outer, 0)
        plsc.subcore_barrier()

        pltpu.sync_copy(s_sh.at[pl.ds(row0, ROWS_PER_TILE)], zb_v)
        pltpu.sync_copy(zb_v, s_out.at[pl.ds(row0, ROWS_PER_TILE)])

    return body


def kernel(Q, K, hyperedge_index_0, extended_index, Wq, bq, Wk, bk, Wv, bv, Wo, bo):
    f32 = jnp.float32

    RB = 400
    bq2 = bq.reshape(1, DIM)
    bk2 = bk.reshape(1, DIM)
    bv2 = bv.reshape(1, DIM)
    row_spec = pl.BlockSpec((RB, DIM), lambda i: (i, 0))
    w_spec = pl.BlockSpec((DIM, DIM), lambda i: (0, 0))
    b_spec = pl.BlockSpec((1, DIM), lambda i: (0, 0))
    qp, kp, vp = pl.pallas_call(
        _proj_body,
        grid=(N // RB,),
        in_specs=[row_spec, row_spec, w_spec, b_spec, w_spec, b_spec, w_spec, b_spec],
        out_specs=[row_spec, row_spec, row_spec],
        out_shape=[jax.ShapeDtypeStruct((N, DIM), f32)] * 3,
    )(Q, K, Wq, bq2, Wk, bk2, Wv, bv2)

    ext_flat = extended_index.astype(jnp.int32).reshape(-1)
    he = hyperedge_index_0.astype(jnp.int32)

    mesh = plsc.VectorSubcoreMesh(core_axis_name="c", subcore_axis_name="s")
    gather_k = functools.partial(
        pl.kernel, mesh=mesh,
        compiler_params=pltpu.CompilerParams(needs_layout_passes=False),
        out_type=[jax.ShapeDtypeStruct((E, DIM), f32)] * 3
        + [jax.ShapeDtypeStruct((M, DIM), f32)],
        scratch_types=[
            pltpu.VMEM((CH * 4,), jnp.int32),
            pltpu.VMEM((CH * 4,), jnp.int32),
            pltpu.VMEM((CH,), jnp.int32),
            pltpu.VMEM((CH,), jnp.int32),
            pltpu.VMEM((CH,), jnp.int32),
            pltpu.VMEM((CH,), jnp.int32),
            pltpu.VMEM((CH, DIM), f32),
            pltpu.VMEM((CH, DIM), f32),
            pltpu.VMEM((CH, DIM), f32),
            pltpu.VMEM((CH, DIM), f32),
            pltpu.VMEM((CH, DIM), f32),
            pltpu.VMEM((CH, DIM), f32),
            pltpu.SemaphoreType.DMA,
            pltpu.SemaphoreType.DMA,
            pltpu.SemaphoreType.DMA,
            pltpu.SemaphoreType.DMA,
            pltpu.SemaphoreType.DMA,
        ],
    )(_sc_gather)
    qi, kj, vj, qhe = gather_k(qp, kp, vp, ext_flat, he)

    hmask = jnp.asarray(np.kron(np.eye(H, dtype=np.float32),
                                np.ones((D_HEAD, D_HEAD), np.float32)))
    sel = jnp.asarray(np.array([[1.0 if (j < H and c == D_HEAD * j) else 0.0
                                 for j in range(16)] for c in range(DIM)],
                               np.float32))
    EB = 1000
    e_row = pl.BlockSpec((EB, DIM), lambda i: (i, 0))
    ta, t1 = pl.pallas_call(
        _edge_body,
        grid=(E // EB,),
        in_specs=[e_row, e_row, e_row,
                  pl.BlockSpec((DIM, DIM), lambda i: (0, 0)),
                  pl.BlockSpec((DIM, 16), lambda i: (0, 0))],
        out_specs=[pl.BlockSpec((EB, HW + 16), lambda i: (i, 0)),
                   pl.BlockSpec((EB, HW), lambda i: (i, 0))],
        out_shape=[jax.ShapeDtypeStruct((E, HW + 16), f32),
                   jax.ShapeDtypeStruct((E, HW), f32)],
    )(qi, kj, vj, hmask, sel)

    WA = HW + 16
    z80 = jnp.zeros((ROWS_PER_TILE, WA), f32)
    z64 = jnp.zeros((ROWS_PER_TILE, HW), f32)
    mesh1 = plsc.VectorSubcoreMesh(core_axis_name="c", subcore_axis_name="s",
                                   num_cores=1)
    cpar = pltpu.CompilerParams(use_tc_tiling_on_sc=False,
                                needs_layout_passes=False)

    def make_scatter_call(width):
        return functools.partial(
            pl.kernel, mesh=mesh1, compiler_params=cpar,
            out_type=jax.ShapeDtypeStruct((M_PAD, width), f32),
            scratch_types=[
                pltpu.VMEM((CH * 4,), jnp.int32),
                pltpu.VMEM((CH * 4,), jnp.int32),
                pltpu.VMEM((CH,), jnp.int32),
                pltpu.VMEM((CH, width), f32),
                pltpu.VMEM((CH, width), f32),
                pltpu.VMEM((ROWS_PER_TILE, width), f32),
                pltpu.VMEM_SHARED((M_PAD, width), f32),
                pltpu.SemaphoreType.DMA,
                pltpu.SemaphoreType.DMA,
                pltpu.SemaphoreType.DMA,
                pltpu.SemaphoreType.DMA,
                pltpu.SemaphoreType.DMA,
            ],
        )(_make_scatter(width))

    sd_flat = make_scatter_call(WA)(ta, ext_flat, z80)
    s1_flat = make_scatter_call(HW)(t1, ext_flat, z64)

    perm = np.zeros((DIM, DIM), np.float32)
    for h in range(H):
        for dd in range(D_HEAD):
            perm[h * D_HEAD + dd, dd * H + h] = 1.0
    pm = jnp.asarray(perm)
    expand = jnp.asarray(np.array(
        [[1.0 if (j < H and c // D_HEAD == j) else 0.0 for c in range(DIM)]
         for j in range(16)], np.float32))
    bo2 = bo.reshape(1, DIM)
    MB = 1000
    m_row = pl.BlockSpec((MB, DIM), lambda i: (i, 0))
    out = pl.pallas_call(
        _final_body,
        grid=(M // MB,),
        in_specs=[pl.BlockSpec((MB, HW + 16), lambda i: (i, 0)),
                  pl.BlockSpec((MB, HW), lambda i: (i, 0)),
                  m_row,
                  pl.BlockSpec((DIM, DIM), lambda i: (0, 0)),
                  pl.BlockSpec((16, DIM), lambda i: (0, 0)),
                  pl.BlockSpec((DIM, DIM), lambda i: (0, 0)),
                  pl.BlockSpec((1, DIM), lambda i: (0, 0))],
        out_specs=m_row,
        out_shape=jax.ShapeDtypeStruct((M, DIM), f32),
    )(sd_flat, s1_flat, qhe, pm, expand, Wo, bo2)

    return out


# speedup vs baseline: 32.8877x; 1.0877x over previous
import functools
import math

import jax
import jax.numpy as jnp
import numpy as np
from jax import lax
from jax.experimental import pallas as pl
from jax.experimental.pallas import tpu as pltpu
from jax.experimental.pallas import tpu_sc as plsc

N = 10000
E = 320000
DIM = 128
H = 8
D_HEAD = DIM // H
M = 10000
INV_SQRT = 1.0 / math.sqrt(DIM)

NC, NS = 2, 16
NW = NC * NS
E_PER_W = E // NW
CH = 80
NCH = E_PER_W // CH
ROWS_PER_TILE = 632
M_PAD = NS * ROWS_PER_TILE
E_PER_TILE = E // NS
HW = DIM // 2


def _proj_body(q_ref, k_ref, wq_ref, bq_ref, wk_ref, bk_ref, wv_ref, bv_ref,
               qp_ref, kp_ref, vp_ref):
    dn = (((1,), (1,)), ((), ()))
    q = q_ref[...]
    k = k_ref[...]
    qp_ref[...] = lax.dot_general(q, wq_ref[...], dn,
                                  preferred_element_type=jnp.float32) + bq_ref[...]
    kp_ref[...] = lax.dot_general(k, wk_ref[...], dn,
                                  preferred_element_type=jnp.float32) + bk_ref[...]
    vp_ref[...] = lax.dot_general(k, wv_ref[...], dn,
                                  preferred_element_type=jnp.float32) + bv_ref[...]


def _edge_body(qi_ref, kj_ref, vj_ref, hmask_ref, sel_ref,
               ta_ref, t1_ref):
    prod = qi_ref[...] * kj_ref[...]
    dots = lax.dot_general(prod, hmask_ref[...], (((1,), (0,)), ((), ())),
                           preferred_element_type=jnp.float32)
    ex = jnp.exp(dots * INV_SQRT)
    t = ex * vj_ref[...]
    e16 = lax.dot_general(ex, sel_ref[...], (((1,), (0,)), ((), ())),
                          preferred_element_type=jnp.float32)
    ta_ref[...] = jnp.concatenate([t[:, :HW], e16], axis=1)
    t1_ref[...] = t[:, HW:]


def _final_body(sd_ref, sp1_ref, qhe_ref, pm_ref, expand_ref,
                wo_ref, bo_ref, o_ref):
    sd = sd_ref[...]
    s = jnp.concatenate([sd[:, :HW], sp1_ref[...]], axis=1)
    d16 = sd[:, HW:]
    d128 = lax.dot_general(d16, expand_ref[...], (((1,), (0,)), ((), ())),
                           preferred_element_type=jnp.float32)
    safe = jnp.where(d128 > 0.0, d128, 1.0)
    qkv = jnp.where(d128 > 0.0, s / safe, 0.0)
    x = qhe_ref[...] + qkv
    o_pre = lax.dot_general(x, pm_ref[...], (((1,), (0,)), ((), ())),
                            preferred_element_type=jnp.float32)
    lin = lax.dot_general(o_pre, wo_ref[...], (((1,), (1,)), ((), ())),
                          preferred_element_type=jnp.float32) + bo_ref[...]
    o_ref[...] = o_pre + jnp.maximum(lin, 0.0)


def _extract_col(ext_v, idx_v, col):
    lanes = lax.iota(jnp.int32, 16)
    for k in range(CH // 16):
        vals = plsc.load_gather(ext_v, [(lanes + 16 * k) * 4 + col])
        idx_v[pl.ds(16 * k, 16)] = vals


def _sc_gather(qp_hbm, kp_hbm, vp_hbm, ext_hbm, he_hbm,
               qi_out, kj_out, vj_out, qhe_out,
               ext0_v, ext1_v, iq0_v, iq1_v, ik0_v, ik1_v,
               rq0_v, rq1_v, rk0_v, rk1_v, rv0_v, rv1_v,
               semE0, semE1, semW0, semW1, sem):
    c = lax.axis_index("c")
    s = lax.axis_index("s")
    wid = s * NC + c
    base0 = wid * E_PER_W

    ext_v = (ext0_v, ext1_v)
    iq_v = (iq0_v, iq1_v)
    ik_v = (ik0_v, ik1_v)
    rq_v = (rq0_v, rq1_v)
    rk_v = (rk0_v, rk1_v)
    rv_v = (rv0_v, rv1_v)
    semE = (semE0, semE1)
    semW = (semW0, semW1)

    def start_ext(i, b):
        base = pl.multiple_of(base0 + i * CH, 8)
        pltpu.async_copy(ext_hbm.at[pl.ds(base * 4, CH * 4)], ext_v[b],
                         semE[b])

    def wait_writebacks(i2, b):
        base = pl.multiple_of(base0 + i2 * CH, 8)
        pltpu.make_async_copy(rq_v[b], qi_out.at[pl.ds(base, CH)],
                              semW[b]).wait()
        pltpu.make_async_copy(rk_v[b], kj_out.at[pl.ds(base, CH)],
                              semW[b]).wait()
        pltpu.make_async_copy(rv_v[b], vj_out.at[pl.ds(base, CH)],
                              semW[b]).wait()

    def proc(i, b, g):
        base = pl.multiple_of(base0 + i * CH, 8)
        pltpu.make_async_copy(ext_hbm.at[pl.ds(base * 4, CH * 4)], ext_v[b],
                              semE[b]).wait()
        _extract_col(ext_v[b], iq_v[b], 0)
        _extract_col(ext_v[b], ik_v[b], 1)

        @pl.when(g > 0)
        def _():
            wait_writebacks(i - 2, b)

        hq = pltpu.async_copy(qp_hbm.at[iq_v[b]], rq_v[b], sem)
        hk = pltpu.async_copy(kp_hbm.at[ik_v[b]], rk_v[b], sem)
        hv = pltpu.async_copy(vp_hbm.at[ik_v[b]], rv_v[b], sem)
        hq.wait()
        hk.wait()
        hv.wait()
        pltpu.async_copy(rq_v[b], qi_out.at[pl.ds(base, CH)], semW[b])
        pltpu.async_copy(rk_v[b], kj_out.at[pl.ds(base, CH)], semW[b])
        pltpu.async_copy(rv_v[b], vj_out.at[pl.ds(base, CH)], semW[b])

    start_ext(0, 0)

    def outer(g, carry):
        i = g * 2
        start_ext(i + 1, 1)
        proc(i, 0, g)

        @pl.when(g < NCH // 2 - 1)
        def _():
            start_ext(i + 2, 0)

        proc(i + 1, 1, g)
        return carry

    lax.fori_loop(0, NCH // 2, outer, 0)
    if NCH % 2:
        start_ext(NCH - 1, 0)
        proc(NCH - 1, 0, 1)
        wait_writebacks(NCH - 2, 1)
        wait_writebacks(NCH - 1, 0)
    else:
        wait_writebacks(NCH - 2, 0)
        wait_writebacks(NCH - 1, 1)

    @pl.when(wid < 25)
    def _():
        def hchunk(i, carry):
            base = pl.multiple_of(wid * 400 + i * CH, 8)
            pltpu.sync_copy(he_hbm.at[pl.ds(base, CH)], iq0_v)
            pltpu.async_copy(qp_hbm.at[iq0_v], rq0_v, sem).wait()
            pltpu.sync_copy(rq0_v, qhe_out.at[pl.ds(base, CH)])
            return carry
        lax.fori_loop(0, 400 // CH, hchunk, 0)


def _make_scatter(width):
    def body(t_hbm, ext_hbm, z_hbm, s_out,
             ext0_v, ext1_v, idx_v, t0_v, t1_v, zb_v, s_sh,
             semE0, semE1, semT0, semT1, sem):
        s = lax.axis_index("s")
        pltpu.sync_copy(z_hbm, zb_v)
        row0 = s * ROWS_PER_TILE
        pltpu.sync_copy(zb_v, s_sh.at[pl.ds(row0, ROWS_PER_TILE)])
        plsc.subcore_barrier()

        base0 = s * E_PER_TILE
        ext_v = (ext0_v, ext1_v)
        t_v = (t0_v, t1_v)
        semE = (semE0, semE1)
        semT = (semT0, semT1)
        nch = E_PER_TILE // CH

        def start(i, b):
            base = pl.multiple_of(base0 + i * CH, 8)
            pltpu.async_copy(ext_hbm.at[pl.ds(base * 4, CH * 4)], ext_v[b],
                             semE[b])
            pltpu.async_copy(t_hbm.at[pl.ds(base, CH)], t_v[b], semT[b])

        def finish(i, b):
            base = pl.multiple_of(base0 + i * CH, 8)
            pltpu.make_async_copy(ext_hbm.at[pl.ds(base * 4, CH * 4)],
                                  ext_v[b], semE[b]).wait()
            pltpu.make_async_copy(t_hbm.at[pl.ds(base, CH)], t_v[b],
                                  semT[b]).wait()
            _extract_col(ext_v[b], idx_v, 3)
            pltpu.sync_copy(t_v[b], s_sh.at[idx_v], add=True)

        start(0, 0)

        def outer(g, carry):
            i = g * 2
            start(i + 1, 1)
            finish(i, 0)

            @pl.when(g < nch // 2 - 1)
            def _():
                start(i + 2, 0)

            finish(i + 1, 1)
            return carry

        lax.fori_loop(0, nch // 2, outer, 0)
        plsc.subcore_barrier()

        pltpu.sync_copy(s_sh.at[pl.ds(row0, ROWS_PER_TILE)], zb_v)
        pltpu.sync_copy(zb_v, s_out.at[pl.ds(row0, ROWS_PER_TILE)])

    return body


def kernel(Q, K, hyperedge_index_0, extended_index, Wq, bq, Wk, bk, Wv, bv, Wo, bo):
    f32 = jnp.float32

    RB = 400
    bq2 = bq.reshape(1, DIM)
    bk2 = bk.reshape(1, DIM)
    bv2 = bv.reshape(1, DIM)
    row_spec = pl.BlockSpec((RB, DIM), lambda i: (i, 0))
    w_spec = pl.BlockSpec((DIM, DIM), lambda i: (0, 0))
    b_spec = pl.BlockSpec((1, DIM), lambda i: (0, 0))
    qp, kp, vp = pl.pallas_call(
        _proj_body,
        grid=(N // RB,),
        in_specs=[row_spec, row_spec, w_spec, b_spec, w_spec, b_spec, w_spec, b_spec],
        out_specs=[row_spec, row_spec, row_spec],
        out_shape=[jax.ShapeDtypeStruct((N, DIM), f32)] * 3,
    )(Q, K, Wq, bq2, Wk, bk2, Wv, bv2)

    ext_flat = extended_index.astype(jnp.int32).reshape(-1)
    he = hyperedge_index_0.astype(jnp.int32)

    mesh = plsc.VectorSubcoreMesh(core_axis_name="c", subcore_axis_name="s")
    gather_k = functools.partial(
        pl.kernel, mesh=mesh,
        compiler_params=pltpu.CompilerParams(needs_layout_passes=False),
        out_type=[jax.ShapeDtypeStruct((E, DIM), f32)] * 3
        + [jax.ShapeDtypeStruct((M, DIM), f32)],
        scratch_types=[
            pltpu.VMEM((CH * 4,), jnp.int32),
            pltpu.VMEM((CH * 4,), jnp.int32),
            pltpu.VMEM((CH,), jnp.int32),
            pltpu.VMEM((CH,), jnp.int32),
            pltpu.VMEM((CH,), jnp.int32),
            pltpu.VMEM((CH,), jnp.int32),
            pltpu.VMEM((CH, DIM), f32),
            pltpu.VMEM((CH, DIM), f32),
            pltpu.VMEM((CH, DIM), f32),
            pltpu.VMEM((CH, DIM), f32),
            pltpu.VMEM((CH, DIM), f32),
            pltpu.VMEM((CH, DIM), f32),
            pltpu.SemaphoreType.DMA,
            pltpu.SemaphoreType.DMA,
            pltpu.SemaphoreType.DMA,
            pltpu.SemaphoreType.DMA,
            pltpu.SemaphoreType.DMA,
        ],
    )(_sc_gather)
    qi, kj, vj, qhe = gather_k(qp, kp, vp, ext_flat, he)

    hmask = jnp.asarray(np.kron(np.eye(H, dtype=np.float32),
                                np.ones((D_HEAD, D_HEAD), np.float32)))
    sel = jnp.asarray(np.array([[1.0 if (j < H and c == D_HEAD * j) else 0.0
                                 for j in range(16)] for c in range(DIM)],
                               np.float32))
    EB = 2000
    e_row = pl.BlockSpec((EB, DIM), lambda i: (i, 0))
    ta, t1 = pl.pallas_call(
        _edge_body,
        grid=(E // EB,),
        in_specs=[e_row, e_row, e_row,
                  pl.BlockSpec((DIM, DIM), lambda i: (0, 0)),
                  pl.BlockSpec((DIM, 16), lambda i: (0, 0))],
        out_specs=[pl.BlockSpec((EB, HW + 16), lambda i: (i, 0)),
                   pl.BlockSpec((EB, HW), lambda i: (i, 0))],
        out_shape=[jax.ShapeDtypeStruct((E, HW + 16), f32),
                   jax.ShapeDtypeStruct((E, HW), f32)],
    )(qi, kj, vj, hmask, sel)

    WA = HW + 16
    z80 = jnp.zeros((ROWS_PER_TILE, WA), f32)
    z64 = jnp.zeros((ROWS_PER_TILE, HW), f32)
    mesh1 = plsc.VectorSubcoreMesh(core_axis_name="c", subcore_axis_name="s",
                                   num_cores=1)
    cpar = pltpu.CompilerParams(use_tc_tiling_on_sc=False,
                                needs_layout_passes=False)

    def make_scatter_call(width):
        return functools.partial(
            pl.kernel, mesh=mesh1, compiler_params=cpar,
            out_type=jax.ShapeDtypeStruct((M_PAD, width), f32),
            scratch_types=[
                pltpu.VMEM((CH * 4,), jnp.int32),
                pltpu.VMEM((CH * 4,), jnp.int32),
                pltpu.VMEM((CH,), jnp.int32),
                pltpu.VMEM((CH, width), f32),
                pltpu.VMEM((CH, width), f32),
                pltpu.VMEM((ROWS_PER_TILE, width), f32),
                pltpu.VMEM_SHARED((M_PAD, width), f32),
                pltpu.SemaphoreType.DMA,
                pltpu.SemaphoreType.DMA,
                pltpu.SemaphoreType.DMA,
                pltpu.SemaphoreType.DMA,
                pltpu.SemaphoreType.DMA,
            ],
        )(_make_scatter(width))

    sd_flat = make_scatter_call(WA)(ta, ext_flat, z80)
    s1_flat = make_scatter_call(HW)(t1, ext_flat, z64)

    perm = np.zeros((DIM, DIM), np.float32)
    for h in range(H):
        for dd in range(D_HEAD):
            perm[h * D_HEAD + dd, dd * H + h] = 1.0
    pm = jnp.asarray(perm)
    expand = jnp.asarray(np.array(
        [[1.0 if (j < H and c // D_HEAD == j) else 0.0 for c in range(DIM)]
         for j in range(16)], np.float32))
    bo2 = bo.reshape(1, DIM)
    MB = 1000
    m_row = pl.BlockSpec((MB, DIM), lambda i: (i, 0))
    out = pl.pallas_call(
        _final_body,
        grid=(M // MB,),
        in_specs=[pl.BlockSpec((MB, HW + 16), lambda i: (i, 0)),
                  pl.BlockSpec((MB, HW), lambda i: (i, 0)),
                  m_row,
                  pl.BlockSpec((DIM, DIM), lambda i: (0, 0)),
                  pl.BlockSpec((16, DIM), lambda i: (0, 0)),
                  pl.BlockSpec((DIM, DIM), lambda i: (0, 0)),
                  pl.BlockSpec((1, DIM), lambda i: (0, 0))],
        out_specs=m_row,
        out_shape=jax.ShapeDtypeStruct((M, DIM), f32),
    )(sd_flat, s1_flat, qhe, pm, expand, Wo, bo2)

    return out
